# Initial kernel scaffold; baseline (speedup 1.0000x reference)
import jax
import jax.numpy as jnp
from jax.experimental import pallas as pl


def kernel(points, params):
    raise NotImplementedError("write your pallas kernel here")


# speedup vs baseline: 10.9048x; 10.9048x over previous
import functools

import jax
import jax.numpy as jnp
import numpy as np
from jax import lax
from jax.experimental import pallas as pl
from jax.experimental.pallas import tpu as pltpu
from jax.experimental.pallas import tpu_sc as plsc

HID = 256; GLOB = 512; LOW = 512; MID = 1024; HIGH = 2048
K = 32; RES = 64; PD = 6; N = 4096
F32 = jnp.float32
BIG = 3.0e38
TQ = 256
SUBTOT = LOW + MID + HIGH
ENCW = 288


def _gelu(x):
    return 0.5 * x * (1.0 + jax.lax.erf(x * np.float32(1.0 / np.sqrt(2.0))))


def _sigm(x):
    return 1.0 / (1.0 + jnp.exp(-x))


def _lnorm(x, g, b):
    m = jnp.mean(x, axis=-1, keepdims=True)
    v = jnp.mean((x - m) * (x - m), axis=-1, keepdims=True)
    return (x - m) / jnp.sqrt(v + 1e-5) * g + b


def _knn_body(pts_ref, xyzT_ref, idx_ref):
    q = pts_ref[0]
    kT = xyzT_ref[0]
    qx = q[:, 0:3]
    qsq = jnp.sum(qx * qx, axis=1, keepdims=True)
    ksq = jnp.sum(kT * kT, axis=0, keepdims=True)
    dot = jax.lax.dot_general(q, kT, (((1,), (0,)), ((), ())),
                              preferred_element_type=F32)
    key = jnp.maximum(qsq + ksq - 2.0 * dot, 0.0)
    lane = jax.lax.broadcasted_iota(jnp.int32, key.shape, 1)
    for t in range(K + 1):
        m = jnp.min(key, axis=1, keepdims=True)
        col = jnp.min(jnp.where(key == m, lane, N), axis=1, keepdims=True)
        if t > 0:
            idx_ref[0, :, t - 1:t] = col
        key = jnp.where(lane == col, BIG, key)


def _knn_call(pts16, xyzT):
    B = pts16.shape[0]
    return pl.pallas_call(
        _knn_body,
        grid=(B, N // TQ),
        in_specs=[
            pl.BlockSpec((1, TQ, 16), lambda b, qt: (b, qt, 0)),
            pl.BlockSpec((1, 16, N), lambda b, qt: (b, 0, 0)),
        ],
        out_specs=pl.BlockSpec((1, TQ, K), lambda b, qt: (b, qt, 0)),
        out_shape=jax.ShapeDtypeStruct((B, N, K), jnp.int32),
    )(pts16, xyzT)


def _enh_body(pts_ref, nb_ref, C1_ref, bc1_ref, C2_ref, bc2_ref,
              N1_ref, bn1_ref, N2_ref, bn2_ref, O1_ref, bo1_ref,
              O2_ref, bo2_ref, out_ref):
    q = pts_ref[0]
    C1 = C1_ref[...]; C2 = C2_ref[...]
    N1 = N1_ref[...]; N2 = N2_ref[...]
    bc1 = bc1_ref[...]; bc2 = bc2_ref[...]
    bn1 = bn1_ref[...]; bn2 = bn2_ref[...]
    qn = q[:, 3:6]
    cf = jnp.full((TQ, HID), -BIG, F32)
    nf = jnp.full((TQ, HID), -BIG, F32)
    for k in range(K):
        nbk = nb_ref[0, :, k, :]
        rel = nbk - q
        h1 = _gelu(jax.lax.dot_general(rel, C1, (((1,), (0,)), ((), ())),
                                       preferred_element_type=F32) + bc1)
        cfk = jax.lax.dot_general(h1, C2, (((1,), (0,)), ((), ())),
                                  preferred_element_type=F32) + bc2
        cf = jnp.maximum(cf, cfk)
        nbn = nbk[:, 3:6]
        var = jnp.abs(1.0 - jnp.sum(qn * nbn, axis=1, keepdims=True))
        feat = jnp.concatenate([nbn, var, jnp.zeros((TQ, 12), F32)], axis=1)
        h2 = _gelu(jax.lax.dot_general(feat, N1, (((1,), (0,)), ((), ())),
                                       preferred_element_type=F32) + bn1)
        nfk = jax.lax.dot_general(h2, N2, (((1,), (0,)), ((), ())),
                                  preferred_element_type=F32) + bn2
        nf = jnp.maximum(nf, nfk)
    cat = jnp.concatenate([cf, nf], axis=1)
    h = _gelu(jax.lax.dot_general(cat, O1_ref[...], (((1,), (0,)), ((), ())),
                                  preferred_element_type=F32) + bo1_ref[...])
    out_ref[0] = jax.lax.dot_general(h, O2_ref[...], (((1,), (0,)), ((), ())),
                                     preferred_element_type=F32) + bo2_ref[...]


def _enh_call(pts16, nb, w):
    B = pts16.shape[0]
    const = lambda shape: pl.BlockSpec(shape, lambda b, qt: tuple(0 for _ in shape))
    return pl.pallas_call(
        _enh_body,
        grid=(B, N // TQ),
        in_specs=[
            pl.BlockSpec((1, TQ, 16), lambda b, qt: (b, qt, 0)),
            pl.BlockSpec((1, TQ, K, 16), lambda b, qt: (b, qt, 0, 0)),
            const((16, HID)), const((1, HID)),
            const((HID, HID)), const((1, HID)),
            const((16, HID)), const((1, HID)),
            const((HID, HID)), const((1, HID)),
            const((2 * HID, HID)), const((1, HID)),
            const((HID, HID)), const((1, HID)),
        ],
        out_specs=pl.BlockSpec((1, TQ, HID), lambda b, qt: (b, qt, 0)),
        out_shape=jax.ShapeDtypeStruct((B, N, HID), F32),
    )(pts16, nb, *w)


_SECTIONS = ((LOW, 0), (MID, LOW), (HIGH, LOW + MID))


def _order_body(xc_ref, xT_ref, out_ref):
    b = pl.program_id(0)
    for Np, off in _SECTIONS:
        xc = xc_ref[0, off:off + Np, 0:3]
        xT = xT_ref[0, 0:3, off:off + Np]
        mn_c = jnp.min(xc, axis=0, keepdims=True)
        mx_c = jnp.max(xc, axis=0, keepdims=True)
        v_c = jnp.clip(((xc - mn_c) / (mx_c - mn_c + 1e-9)
                        * RES).astype(jnp.int32), 0, RES - 1)
        key_c = v_c[:, 0:1] * (RES * RES) + v_c[:, 1:2] * RES + v_c[:, 2:3]
        iota_c = jax.lax.broadcasted_iota(jnp.int32, (Np, 1), 0)
        comb_c = key_c * Np + iota_c
        mn_r = jnp.min(xT, axis=1, keepdims=True)
        mx_r = jnp.max(xT, axis=1, keepdims=True)
        v_r = jnp.clip(((xT - mn_r) / (mx_r - mn_r + 1e-9)
                        * RES).astype(jnp.int32), 0, RES - 1)
        key_r = v_r[0:1, :] * (RES * RES) + v_r[1:2, :] * RES + v_r[2:3, :]
        iota_r = jax.lax.broadcasted_iota(jnp.int32, (1, Np), 1)
        comb_r = key_r * Np + iota_r
        rank = jnp.zeros((Np, 1), jnp.int32)
        CH = 512
        for j0 in range(0, Np, CH):
            cmp = (comb_r[:, j0:j0 + CH] < comb_c).astype(jnp.int32)
            rank = rank + jnp.sum(cmp, axis=1, keepdims=True)
        orow = jnp.zeros((1, Np), jnp.int32)
        for i0 in range(0, Np, CH):
            rc = rank[i0:i0 + CH]
            sel = rc == iota_r
            idx_i = iota_c[i0:i0 + CH]
            orow = orow + jnp.sum(jnp.where(sel, idx_i, 0), axis=0,
                                  keepdims=True)
        out_ref[0, :, off:off + Np] = orow + (b * SUBTOT + off)


def _order_call(xc, xT):
    B = xc.shape[0]
    return pl.pallas_call(
        _order_body,
        grid=(B,),
        in_specs=[
            pl.BlockSpec((1, SUBTOT, 16), lambda b: (b, 0, 0)),
            pl.BlockSpec((1, 16, SUBTOT), lambda b: (b, 0, 0)),
        ],
        out_specs=pl.BlockSpec((1, 1, SUBTOT), lambda b: (b, 0, 0)),
        out_shape=jax.ShapeDtypeStruct((B, 1, SUBTOT), jnp.int32),
    )(xc, xT)


def _ema_scan(b0, a, L, reverse):
    S = b0
    p = a
    sh = 1
    while sh < L:
        if reverse:
            shifted = jnp.concatenate(
                [S[sh:], jnp.zeros((sh, HID), F32)], axis=0)
        else:
            shifted = jnp.concatenate(
                [jnp.zeros((sh, HID), F32), S[:L - sh]], axis=0)
        S = S + p * shifted
        p = p * p
        sh *= 2
    return S


def _encoder_body(Np, ord_ref, Wp_ref, bp_ref, Wo_ref, bo_ref,
                  LNW_ref, LNB_ref, WIN_ref, BIN_ref, AF_ref, AB_ref,
                  WOUT_ref, BOUT_ref, LFW_ref, LFB_ref,
                  O1_ref, b1_ref, O2_ref, b2_ref, out_ref):
    L = Np + 2
    xin = ord_ref[0]
    x = jax.lax.dot_general(xin, Wp_ref[...], (((1,), (0,)), ((), ())),
                            preferred_element_type=F32) + bp_ref[...]
    s_tok = bo_ref[...]
    e_tok = Wo_ref[...] + bo_ref[...]
    for i in range(4):
        seq = jnp.concatenate([s_tok, x, e_tok], axis=0)
        h = _lnorm(seq, LNW_ref[i:i + 1], LNB_ref[i:i + 1])
        u = jax.lax.dot_general(h, WIN_ref[i], (((1,), (0,)), ((), ())),
                                preferred_element_type=F32) + BIN_ref[i:i + 1]
        u1 = u[:, :HID]
        g = u[:, HID:]
        af = _sigm(AF_ref[i:i + 1])
        ab = _sigm(AB_ref[i:i + 1])
        sf = _ema_scan(u1 * (1.0 - af), af, L, reverse=False)
        sb = _ema_scan(u1 * (1.0 - ab), ab, L, reverse=True)
        y = (sf + sb) * (g * _sigm(g))
        xn = seq + jax.lax.dot_general(y, WOUT_ref[i], (((1,), (0,)), ((), ())),
                                       preferred_element_type=F32) + BOUT_ref[i:i + 1]
        x = xn[1:1 + Np]
    f = _lnorm(x, LFW_ref[...], LFB_ref[...])
    mx = jnp.max(f, axis=0, keepdims=True)
    mean = jnp.sum(f, axis=0, keepdims=True) * np.float32(1.0 / Np)
    pooled = jnp.concatenate([mx, mean], axis=1)
    h2 = _gelu(jax.lax.dot_general(pooled, O1_ref[...], (((1,), (0,)), ((), ())),
                                   preferred_element_type=F32) + b1_ref[...])
    out_ref[0] = jax.lax.dot_general(h2, O2_ref[...], (((1,), (0,)), ((), ())),
                                     preferred_element_type=F32) + b2_ref[...]


def _encoder_call(ordered, w, Np):
    B = ordered.shape[0]
    shapes = [(ENCW, HID), (1, HID), (1, HID), (1, HID),
              (4, HID), (4, HID), (4, HID, 2 * HID), (4, 2 * HID),
              (4, HID), (4, HID), (4, HID, HID), (4, HID),
              (1, HID), (1, HID),
              (2 * HID, GLOB), (1, GLOB), (GLOB, GLOB), (1, GLOB)]
    const = lambda shape: pl.BlockSpec(shape, lambda b: tuple(0 for _ in shape))
    return pl.pallas_call(
        functools.partial(_encoder_body, Np),
        grid=(B,),
        in_specs=[pl.BlockSpec((1, Np, ENCW), lambda b: (b, 0, 0))]
        + [const(s) for s in shapes],
        out_specs=pl.BlockSpec((1, 1, GLOB), lambda b: (b, 0, 0)),
        out_shape=jax.ShapeDtypeStruct((B, 1, GLOB), F32),
    )(ordered, *w)[:, 0]


def _heads_body(F_ref, W11, b11, W12, b12, W21, b21, W22, b22,
                W31, b31, W32, b32, WL, bL, WM, bM, WH, bH,
                lo_ref, mi_ref, hi_ref):
    F = F_ref[...]
    def mlp2(w1, c1, w2, c2):
        h = _gelu(jax.lax.dot_general(F, w1[...], (((1,), (0,)), ((), ())),
                                      preferred_element_type=F32) + c1[...])
        return jax.lax.dot_general(h, w2[...], (((1,), (0,)), ((), ())),
                                   preferred_element_type=F32) + c2[...]
    f1 = mlp2(W11, b11, W12, b12)
    f2 = mlp2(W21, b21, W22, b22)
    f3 = mlp2(W31, b31, W32, b32)
    lo_ref[...] = jax.lax.dot_general(f3, WL[...], (((1,), (0,)), ((), ())),
                                      preferred_element_type=F32) + bL[...]
    mi_ref[...] = jax.lax.dot_general(f2, WM[...], (((1,), (0,)), ((), ())),
                                      preferred_element_type=F32) + bM[...]
    hi_ref[...] = jax.lax.dot_general(f1, WH[...], (((1,), (0,)), ((), ())),
                                      preferred_element_type=F32) + bH[...]


def _heads_call(F, w):
    B = F.shape[0]
    return pl.pallas_call(
        _heads_body,
        out_shape=[jax.ShapeDtypeStruct((B, LOW * 3), F32),
                   jax.ShapeDtypeStruct((B, MID * 3), F32),
                   jax.ShapeDtypeStruct((B, HIGH * 3), F32)],
    )(F, *w)


def _sc_gather(table, idx, chunk):
    nwork, nch, _ = idx.shape
    W = table.shape[1]
    mesh = plsc.VectorSubcoreMesh(core_axis_name="c", subcore_axis_name="s")
    nc = mesh.num_cores

    @functools.partial(
        pl.kernel,
        out_type=jax.ShapeDtypeStruct((nwork * nch * chunk, W), F32),
        mesh=mesh,
        scratch_types=[
            pltpu.VMEM((chunk,), jnp.int32),
            pltpu.VMEM((chunk, W), F32),
            pltpu.SemaphoreType.DMA,
        ],
    )
    def gath(idx_hbm, table_hbm, out_hbm, idx_v, rows_v, sem):
        wid = lax.axis_index("s") * nc + lax.axis_index("c")

        def body(c, _):
            pltpu.sync_copy(idx_hbm.at[wid, c], idx_v)
            pltpu.async_copy(table_hbm.at[idx_v], rows_v, sem).wait()
            row0 = (wid * nch + c) * chunk
            pltpu.sync_copy(rows_v, out_hbm.at[pl.ds(row0, chunk)])
            return ()

        lax.fori_loop(0, nch, body, (), unroll=False)

    return gath(idx, table)


def _row(v):
    return v.reshape(1, -1)


def _pad_rows(W, rows):
    return jnp.concatenate(
        [W, jnp.zeros((rows - W.shape[0], W.shape[1]), F32)], axis=0)


def kernel(points, params):
    B = points.shape[0]
    pts16 = jnp.concatenate(
        [points, jnp.zeros((B, N, 16 - PD), F32)], axis=-1)
    xyzT = jnp.concatenate(
        [jnp.swapaxes(points[..., :3], 1, 2),
         jnp.zeros((B, 13, N), F32)], axis=1)

    idx = _knn_call(pts16, xyzT)
    nb = jnp.take_along_axis(
        pts16, idx.reshape(B, N * K)[..., None], axis=1).reshape(B, N, K, 16)
    e = params['enh']
    enh_w = (_pad_rows(e['coord1'][0], 16), _row(e['coord1'][1]),
             e['coord2'][0], _row(e['coord2'][1]),
             _pad_rows(e['norm1'][0], 16), _row(e['norm1'][1]),
             e['norm2'][0], _row(e['norm2'][1]),
             e['out1'][0], _row(e['out1'][1]),
             e['out2'][0], _row(e['out2'][1]))
    enh = _enh_call(pts16, nb, enh_w)

    enc_in = jnp.concatenate(
        [points, enh, jnp.zeros((B, N, ENCW - PD - HID), F32)], axis=-1)
    subs = [enc_in[:, ::N // m][:, :m] for m in (LOW, MID, HIGH)]
    comb = jnp.concatenate(subs, axis=1)
    order = _order_call(comb[..., :16],
                        jnp.swapaxes(comb[..., :16], 1, 2))
    comb_flat = comb.reshape(B * SUBTOT, ENCW)
    ordered = jnp.take(comb_flat, order.reshape(-1), axis=0)
    ordered = ordered.reshape(B, SUBTOT, ENCW)

    def enc_w(p):
        bl = p['blocks']
        st = lambda key: jnp.stack([b[key] for b in bl])
        return (_pad_rows(p['in_proj'][0], ENCW), _row(p['in_proj'][1]),
                p['oip'][0], _row(p['oip'][1]),
                jnp.stack([b['ln'][0] for b in bl]),
                jnp.stack([b['ln'][1] for b in bl]),
                st('W_in'), st('b_in'), st('a_fwd'), st('a_bwd'),
                jnp.stack([b['out'][0] for b in bl]),
                jnp.stack([b['out'][1] for b in bl]),
                _row(p['ln'][0]), _row(p['ln'][1]),
                p['out1'][0], _row(p['out1'][1]),
                p['out2'][0], _row(p['out2'][1]))

    fl = _encoder_call(ordered[:, :LOW], enc_w(params['enc_low']), LOW)
    fm = _encoder_call(ordered[:, LOW:LOW + MID], enc_w(params['enc_mid']), MID)
    fh = _encoder_call(ordered[:, LOW + MID:], enc_w(params['enc_high']), HIGH)

    F = jnp.concatenate([fl, fm, fh], axis=-1)
    hw = []
    for g in ('g2f1', 'g2f2', 'g2f3'):
        hw += [params[g]['l1'][0], _row(params[g]['l1'][1]),
               params[g]['l2'][0], _row(params[g]['l2'][1])]
    for d in ('low', 'mid', 'high'):
        hw += [params['dec'][d][0], _row(params['dec'][d][1])]
    lo, mi, hi = _heads_call(F, hw)
    p_lo = lo.reshape(B, LOW, 3)
    p_mi = jnp.repeat(p_lo, 2, axis=1) + mi.reshape(B, MID, 3)
    p_hi = jnp.repeat(p_mi, 2, axis=1) + hi.reshape(B, HIGH, 3)
    return (p_lo, p_mi, p_hi)


# speedup vs baseline: 22.4372x; 2.0576x over previous
import functools

import jax
import jax.numpy as jnp
import numpy as np
from jax import lax
from jax.experimental import pallas as pl
from jax.experimental.pallas import tpu as pltpu
from jax.experimental.pallas import tpu_sc as plsc

HID = 256; GLOB = 512; LOW = 512; MID = 1024; HIGH = 2048
K = 32; RES = 64; PD = 6; N = 4096
F32 = jnp.float32
BIG = 3.0e38
TQ = 256
SUBTOT = LOW + MID + HIGH
ENCW = 288


def _gelu(x):
    return 0.5 * x * (1.0 + jax.lax.erf(x * np.float32(1.0 / np.sqrt(2.0))))


def _sigm(x):
    return 1.0 / (1.0 + jnp.exp(-x))


def _lnorm(x, g, b):
    m = jnp.mean(x, axis=-1, keepdims=True)
    v = jnp.mean((x - m) * (x - m), axis=-1, keepdims=True)
    return (x - m) / jnp.sqrt(v + 1e-5) * g + b


def _knn_body(pts_ref, xyzT_ref, idx_ref):
    q = pts_ref[0]
    kT = xyzT_ref[0]
    qx = q[:, 0:3]
    qsq = jnp.sum(qx * qx, axis=1, keepdims=True)
    ksq = jnp.sum(kT * kT, axis=0, keepdims=True)
    dot = jax.lax.dot_general(q, kT, (((1,), (0,)), ((), ())),
                              preferred_element_type=F32)
    key = jnp.maximum(qsq + ksq - 2.0 * dot, 0.0)
    lane = jax.lax.broadcasted_iota(jnp.int32, key.shape, 1)
    base = pl.program_id(0) * N
    for t in range(K + 1):
        m = jnp.min(key, axis=1, keepdims=True)
        col = jnp.min(jnp.where(key == m, lane, N), axis=1, keepdims=True)
        if t > 0:
            idx_ref[0, :, t - 1:t] = col + base
        key = jnp.where(lane == col, BIG, key)


def _knn_call(pts16, xyzT):
    B = pts16.shape[0]
    return pl.pallas_call(
        _knn_body,
        grid=(B, N // TQ),
        in_specs=[
            pl.BlockSpec((1, TQ, 16), lambda b, qt: (b, qt, 0)),
            pl.BlockSpec((1, 16, N), lambda b, qt: (b, 0, 0)),
        ],
        out_specs=pl.BlockSpec((1, TQ, K), lambda b, qt: (b, qt, 0)),
        out_shape=jax.ShapeDtypeStruct((B, N, K), jnp.int32),
    )(pts16, xyzT)


def _enh_body(pts_ref, nb_ref, C1_ref, bc1_ref, C2_ref, bc2_ref,
              N1_ref, bn1_ref, N2_ref, bn2_ref, O1_ref, bo1_ref,
              O2_ref, bo2_ref, out_ref):
    q = pts_ref[0]
    C1 = C1_ref[...]; C2 = C2_ref[...]
    N1 = N1_ref[...]; N2 = N2_ref[...]
    bc1 = bc1_ref[...]; bc2 = bc2_ref[...]
    bn1 = bn1_ref[...]; bn2 = bn2_ref[...]
    qn = q[:, 3:6]
    cf = jnp.full((TQ, HID), -BIG, F32)
    nf = jnp.full((TQ, HID), -BIG, F32)
    for k in range(K):
        nbk = nb_ref[0, :, k, :]
        rel = nbk - q
        h1 = _gelu(jax.lax.dot_general(rel, C1, (((1,), (0,)), ((), ())),
                                       preferred_element_type=F32) + bc1)
        cfk = jax.lax.dot_general(h1, C2, (((1,), (0,)), ((), ())),
                                  preferred_element_type=F32) + bc2
        cf = jnp.maximum(cf, cfk)
        nbn = nbk[:, 3:6]
        var = jnp.abs(1.0 - jnp.sum(qn * nbn, axis=1, keepdims=True))
        feat = jnp.concatenate([nbn, var, jnp.zeros((TQ, 12), F32)], axis=1)
        h2 = _gelu(jax.lax.dot_general(feat, N1, (((1,), (0,)), ((), ())),
                                       preferred_element_type=F32) + bn1)
        nfk = jax.lax.dot_general(h2, N2, (((1,), (0,)), ((), ())),
                                  preferred_element_type=F32) + bn2
        nf = jnp.maximum(nf, nfk)
    cat = jnp.concatenate([cf, nf], axis=1)
    h = _gelu(jax.lax.dot_general(cat, O1_ref[...], (((1,), (0,)), ((), ())),
                                  preferred_element_type=F32) + bo1_ref[...])
    out_ref[0] = jax.lax.dot_general(h, O2_ref[...], (((1,), (0,)), ((), ())),
                                     preferred_element_type=F32) + bo2_ref[...]


def _enh_call(pts16, nb, w):
    B = pts16.shape[0]
    const = lambda shape: pl.BlockSpec(shape, lambda b, qt: tuple(0 for _ in shape))
    return pl.pallas_call(
        _enh_body,
        grid=(B, N // TQ),
        in_specs=[
            pl.BlockSpec((1, TQ, 16), lambda b, qt: (b, qt, 0)),
            pl.BlockSpec((1, TQ, K, 16), lambda b, qt: (b, qt, 0, 0)),
            const((16, HID)), const((1, HID)),
            const((HID, HID)), const((1, HID)),
            const((16, HID)), const((1, HID)),
            const((HID, HID)), const((1, HID)),
            const((2 * HID, HID)), const((1, HID)),
            const((HID, HID)), const((1, HID)),
        ],
        out_specs=pl.BlockSpec((1, TQ, HID), lambda b, qt: (b, qt, 0)),
        out_shape=jax.ShapeDtypeStruct((B, N, HID), F32),
    )(pts16, nb, *w)


_SECTIONS = ((LOW, 0), (MID, LOW), (HIGH, LOW + MID))


def _order_body(xc_ref, xT_ref, out_ref):
    b = pl.program_id(0)
    for Np, off in _SECTIONS:
        xc = xc_ref[0, off:off + Np, 0:3]
        xT = xT_ref[0, 0:3, off:off + Np]
        mn_c = jnp.min(xc, axis=0, keepdims=True)
        mx_c = jnp.max(xc, axis=0, keepdims=True)
        v_c = jnp.clip(((xc - mn_c) / (mx_c - mn_c + 1e-9)
                        * RES).astype(jnp.int32), 0, RES - 1)
        key_c = v_c[:, 0:1] * (RES * RES) + v_c[:, 1:2] * RES + v_c[:, 2:3]
        iota_c = jax.lax.broadcasted_iota(jnp.int32, (Np, 1), 0)
        comb_c = key_c * Np + iota_c
        mn_r = jnp.min(xT, axis=1, keepdims=True)
        mx_r = jnp.max(xT, axis=1, keepdims=True)
        v_r = jnp.clip(((xT - mn_r) / (mx_r - mn_r + 1e-9)
                        * RES).astype(jnp.int32), 0, RES - 1)
        key_r = v_r[0:1, :] * (RES * RES) + v_r[1:2, :] * RES + v_r[2:3, :]
        iota_r = jax.lax.broadcasted_iota(jnp.int32, (1, Np), 1)
        comb_r = key_r * Np + iota_r
        rank = jnp.zeros((Np, 1), jnp.int32)
        CH = 512
        for j0 in range(0, Np, CH):
            cmp = (comb_r[:, j0:j0 + CH] < comb_c).astype(jnp.int32)
            rank = rank + jnp.sum(cmp, axis=1, keepdims=True)
        orow = jnp.zeros((1, Np), jnp.int32)
        for i0 in range(0, Np, CH):
            rc = rank[i0:i0 + CH]
            sel = rc == iota_r
            idx_i = iota_c[i0:i0 + CH]
            orow = orow + jnp.sum(jnp.where(sel, idx_i, 0), axis=0,
                                  keepdims=True)
        out_ref[0, :, off:off + Np] = orow + (b * SUBTOT + off)


def _order_call(xc, xT):
    B = xc.shape[0]
    return pl.pallas_call(
        _order_body,
        grid=(B,),
        in_specs=[
            pl.BlockSpec((1, SUBTOT, 16), lambda b: (b, 0, 0)),
            pl.BlockSpec((1, 16, SUBTOT), lambda b: (b, 0, 0)),
        ],
        out_specs=pl.BlockSpec((1, 1, SUBTOT), lambda b: (b, 0, 0)),
        out_shape=jax.ShapeDtypeStruct((B, 1, SUBTOT), jnp.int32),
    )(xc, xT)


def _ema_scan(b0, a, L, reverse):
    S = b0
    p = a
    sh = 1
    while sh < L:
        if reverse:
            shifted = jnp.concatenate(
                [S[sh:], jnp.zeros((sh, HID), F32)], axis=0)
        else:
            shifted = jnp.concatenate(
                [jnp.zeros((sh, HID), F32), S[:L - sh]], axis=0)
        S = S + p * shifted
        p = p * p
        sh *= 2
    return S


def _encoder_body(Np, ord_ref, Wp_ref, bp_ref, Wo_ref, bo_ref,
                  LNW_ref, LNB_ref, WIN_ref, BIN_ref, AF_ref, AB_ref,
                  WOUT_ref, BOUT_ref, LFW_ref, LFB_ref,
                  O1_ref, b1_ref, O2_ref, b2_ref, out_ref):
    L = Np + 2
    xin = ord_ref[0]
    x = jax.lax.dot_general(xin, Wp_ref[...], (((1,), (0,)), ((), ())),
                            preferred_element_type=F32) + bp_ref[...]
    s_tok = bo_ref[...]
    e_tok = Wo_ref[...] + bo_ref[...]
    for i in range(4):
        seq = jnp.concatenate([s_tok, x, e_tok], axis=0)
        h = _lnorm(seq, LNW_ref[i:i + 1], LNB_ref[i:i + 1])
        u = jax.lax.dot_general(h, WIN_ref[i], (((1,), (0,)), ((), ())),
                                preferred_element_type=F32) + BIN_ref[i:i + 1]
        u1 = u[:, :HID]
        g = u[:, HID:]
        af = _sigm(AF_ref[i:i + 1])
        ab = _sigm(AB_ref[i:i + 1])
        sf = _ema_scan(u1 * (1.0 - af), af, L, reverse=False)
        sb = _ema_scan(u1 * (1.0 - ab), ab, L, reverse=True)
        y = (sf + sb) * (g * _sigm(g))
        xn = seq + jax.lax.dot_general(y, WOUT_ref[i], (((1,), (0,)), ((), ())),
                                       preferred_element_type=F32) + BOUT_ref[i:i + 1]
        x = xn[1:1 + Np]
    f = _lnorm(x, LFW_ref[...], LFB_ref[...])
    mx = jnp.max(f, axis=0, keepdims=True)
    mean = jnp.sum(f, axis=0, keepdims=True) * np.float32(1.0 / Np)
    pooled = jnp.concatenate([mx, mean], axis=1)
    h2 = _gelu(jax.lax.dot_general(pooled, O1_ref[...], (((1,), (0,)), ((), ())),
                                   preferred_element_type=F32) + b1_ref[...])
    out_ref[0] = jax.lax.dot_general(h2, O2_ref[...], (((1,), (0,)), ((), ())),
                                     preferred_element_type=F32) + b2_ref[...]


def _encoder_call(ordered, w, Np):
    B = ordered.shape[0]
    shapes = [(ENCW, HID), (1, HID), (1, HID), (1, HID),
              (4, HID), (4, HID), (4, HID, 2 * HID), (4, 2 * HID),
              (4, HID), (4, HID), (4, HID, HID), (4, HID),
              (1, HID), (1, HID),
              (2 * HID, GLOB), (1, GLOB), (GLOB, GLOB), (1, GLOB)]
    const = lambda shape: pl.BlockSpec(shape, lambda b: tuple(0 for _ in shape))
    return pl.pallas_call(
        functools.partial(_encoder_body, Np),
        grid=(B,),
        in_specs=[pl.BlockSpec((1, Np, ENCW), lambda b: (b, 0, 0))]
        + [const(s) for s in shapes],
        out_specs=pl.BlockSpec((1, 1, GLOB), lambda b: (b, 0, 0)),
        out_shape=jax.ShapeDtypeStruct((B, 1, GLOB), F32),
    )(ordered, *w)[:, 0]


def _heads_body(F_ref, W11, b11, W12, b12, W21, b21, W22, b22,
                W31, b31, W32, b32, WL, bL, WM, bM, WH, bH,
                lo_ref, mi_ref, hi_ref):
    F = F_ref[...]
    def mlp2(w1, c1, w2, c2):
        h = _gelu(jax.lax.dot_general(F, w1[...], (((1,), (0,)), ((), ())),
                                      preferred_element_type=F32) + c1[...])
        return jax.lax.dot_general(h, w2[...], (((1,), (0,)), ((), ())),
                                   preferred_element_type=F32) + c2[...]
    f1 = mlp2(W11, b11, W12, b12)
    f2 = mlp2(W21, b21, W22, b22)
    f3 = mlp2(W31, b31, W32, b32)
    lo_ref[...] = jax.lax.dot_general(f3, WL[...], (((1,), (0,)), ((), ())),
                                      preferred_element_type=F32) + bL[...]
    mi_ref[...] = jax.lax.dot_general(f2, WM[...], (((1,), (0,)), ((), ())),
                                      preferred_element_type=F32) + bM[...]
    hi_ref[...] = jax.lax.dot_general(f1, WH[...], (((1,), (0,)), ((), ())),
                                      preferred_element_type=F32) + bH[...]


def _heads_call(F, w):
    B = F.shape[0]
    return pl.pallas_call(
        _heads_body,
        out_shape=[jax.ShapeDtypeStruct((B, LOW * 3), F32),
                   jax.ShapeDtypeStruct((B, MID * 3), F32),
                   jax.ShapeDtypeStruct((B, HIGH * 3), F32)],
    )(F, *w)


def _sc_gather(table, idx, chunk):
    nwork, nch, _ = idx.shape
    W = table.shape[1]
    mesh = plsc.VectorSubcoreMesh(core_axis_name="c", subcore_axis_name="s")
    nc = mesh.num_cores

    @functools.partial(
        pl.kernel,
        out_type=jax.ShapeDtypeStruct((nwork * nch * chunk, W), F32),
        mesh=mesh,
        compiler_params=pltpu.CompilerParams(use_tc_tiling_on_sc=False),
        scratch_types=[
            pltpu.VMEM((chunk,), jnp.int32),
            pltpu.VMEM((chunk, W), F32),
            pltpu.SemaphoreType.DMA,
        ],
    )
    def gath(idx_hbm, table_hbm, out_hbm, idx_v, rows_v, sem):
        wid = lax.axis_index("s") * nc + lax.axis_index("c")

        def body(c, _):
            pltpu.sync_copy(idx_hbm.at[wid, c], idx_v)
            pltpu.async_copy(table_hbm.at[idx_v], rows_v, sem).wait()
            row0 = (wid * nch + c) * chunk
            pltpu.sync_copy(rows_v, out_hbm.at[pl.ds(row0, chunk)])
            return ()

        lax.fori_loop(0, nch, body, (), unroll=False)

    return gath(idx, table)


def _row(v):
    return v.reshape(1, -1)


def _pad_rows(W, rows):
    return jnp.concatenate(
        [W, jnp.zeros((rows - W.shape[0], W.shape[1]), F32)], axis=0)


def kernel(points, params):
    B = points.shape[0]
    pts16 = jnp.concatenate(
        [points, jnp.zeros((B, N, 16 - PD), F32)], axis=-1)
    xyzT = jnp.concatenate(
        [jnp.swapaxes(points[..., :3], 1, 2),
         jnp.zeros((B, 13, N), F32)], axis=1)

    idx = _knn_call(pts16, xyzT)
    nb = _sc_gather(pts16.reshape(B * N, 16),
                    idx.reshape(32, (B * N * K) // (32 * 128), 128),
                    128).reshape(B, N, K, 16)
    e = params['enh']
    enh_w = (_pad_rows(e['coord1'][0], 16), _row(e['coord1'][1]),
             e['coord2'][0], _row(e['coord2'][1]),
             _pad_rows(e['norm1'][0], 16), _row(e['norm1'][1]),
             e['norm2'][0], _row(e['norm2'][1]),
             e['out1'][0], _row(e['out1'][1]),
             e['out2'][0], _row(e['out2'][1]))
    enh = _enh_call(pts16, nb, enh_w)

    enc_in = jnp.concatenate(
        [points, enh, jnp.zeros((B, N, ENCW - PD - HID), F32)], axis=-1)
    subs = [enc_in[:, ::N // m][:, :m] for m in (LOW, MID, HIGH)]
    comb = jnp.concatenate(subs, axis=1)
    order = _order_call(comb[..., :16],
                        jnp.swapaxes(comb[..., :16], 1, 2))
    ordered = _sc_gather(comb.reshape(B * SUBTOT, ENCW),
                         order.reshape(32, (B * SUBTOT) // (32 * 112), 112),
                         112).reshape(B, SUBTOT, ENCW)

    def enc_w(p):
        bl = p['blocks']
        st = lambda key: jnp.stack([b[key] for b in bl])
        return (_pad_rows(p['in_proj'][0], ENCW), _row(p['in_proj'][1]),
                p['oip'][0], _row(p['oip'][1]),
                jnp.stack([b['ln'][0] for b in bl]),
                jnp.stack([b['ln'][1] for b in bl]),
                st('W_in'), st('b_in'), st('a_fwd'), st('a_bwd'),
                jnp.stack([b['out'][0] for b in bl]),
                jnp.stack([b['out'][1] for b in bl]),
                _row(p['ln'][0]), _row(p['ln'][1]),
                p['out1'][0], _row(p['out1'][1]),
                p['out2'][0], _row(p['out2'][1]))

    fl = _encoder_call(ordered[:, :LOW], enc_w(params['enc_low']), LOW)
    fm = _encoder_call(ordered[:, LOW:LOW + MID], enc_w(params['enc_mid']), MID)
    fh = _encoder_call(ordered[:, LOW + MID:], enc_w(params['enc_high']), HIGH)

    F = jnp.concatenate([fl, fm, fh], axis=-1)
    hw = []
    for g in ('g2f1', 'g2f2', 'g2f3'):
        hw += [params[g]['l1'][0], _row(params[g]['l1'][1]),
               params[g]['l2'][0], _row(params[g]['l2'][1])]
    for d in ('low', 'mid', 'high'):
        hw += [params['dec'][d][0], _row(params['dec'][d][1])]
    lo, mi, hi = _heads_call(F, hw)
    p_lo = lo.reshape(B, LOW, 3)
    p_mi = jnp.repeat(p_lo, 2, axis=1) + mi.reshape(B, MID, 3)
    p_hi = jnp.repeat(p_mi, 2, axis=1) + hi.reshape(B, HIGH, 3)
    return (p_lo, p_mi, p_hi)
